# baseline (speedup 1.0000x reference)
import functools

import jax
import jax.numpy as jnp
from jax import lax
from jax.experimental import pallas as pl
from jax.experimental.pallas import tpu as pltpu
from jax.experimental.pallas import tpu_sc as plsc

BATCH = 16384
FEAT = 32
NC, NS, L = 2, 16, 16
NW = NC * NS
BPW = BATCH // NW
GCH = 128
NG = BPW // GCH


def _sc_partials(features, labels, centers):
    mesh = plsc.VectorSubcoreMesh(core_axis_name="c", subcore_axis_name="s")

    @functools.partial(
        pl.kernel,
        mesh=mesh,
        out_type=jax.ShapeDtypeStruct((NW, L), jnp.float32),
        scratch_types=[
            pltpu.VMEM((NG, GCH), jnp.int32),
            pltpu.VMEM((NG, GCH, FEAT), jnp.float32),
            pltpu.VMEM((NG, GCH, FEAT), jnp.float32),
            pltpu.VMEM((L,), jnp.float32),
            pltpu.SemaphoreType.DMA,
        ],
        compiler_params=pltpu.CompilerParams(use_tc_tiling_on_sc=False),
    )
    def k(feat_hbm, lab_hbm, cent_hbm, out_hbm, idx_v, feat_v, rows_v, acc_v, sem):
        wid = lax.axis_index("s") * NC + lax.axis_index("c")
        pltpu.sync_copy(lab_hbm.at[wid], idx_v)
        cps = [pltpu.make_async_copy(feat_hbm.at[wid], feat_v, sem)]
        cps += [pltpu.make_async_copy(cent_hbm.at[idx_v.at[g]], rows_v.at[g], sem)
                for g in range(NG)]
        for cp in cps:
            cp.start()
        for cp in cps:
            cp.wait()

        def body(i, acc):
            for g in range(NG):
                f0 = feat_v[g, i, pl.ds(0, L)]
                c0 = rows_v[g, i, pl.ds(0, L)]
                f1 = feat_v[g, i, pl.ds(L, L)]
                c1 = rows_v[g, i, pl.ds(L, L)]
                d0 = f0 - c0
                d1 = f1 - c1
                acc = acc + d0 * d0 + d1 * d1
            return acc

        acc = lax.fori_loop(0, GCH, body, jnp.zeros((L,), jnp.float32))
        acc_v[...] = acc
        pltpu.sync_copy(acc_v, out_hbm.at[wid])

    return k(features.reshape(NW, NG, GCH, FEAT),
             labels.reshape(NW, NG, GCH), centers)


def _tc_reduce(partials):
    def body(p_ref, o_ref):
        o_ref[0, 0] = jnp.sum(p_ref[...]) * (1.0 / BATCH)

    out = pl.pallas_call(
        body,
        out_shape=jax.ShapeDtypeStruct((1, 1), jnp.float32),
        out_specs=pl.BlockSpec(memory_space=pltpu.SMEM),
    )(partials)
    return out.reshape(())


def kernel(features, labels, centers):
    labels = labels.astype(jnp.int32)
    partials = _sc_partials(features, labels, centers)
    return _tc_reduce(partials)
